# baseline (device time: 112754 ns/iter reference)
import jax
import jax.numpy as jnp
from jax import lax
from jax.experimental import pallas as pl
from jax.experimental.pallas import tpu as pltpu


def kernel(O, Wo):
    B, S, H, D = O.shape
    HD = H * D
    N = Wo.shape[1]
    S_half = S // 2

    O2 = O.reshape(B, S, HD)

    def body(o_ref, wo_ref, out_ref, sbuf, rbuf, send_sem, recv_sem):
        my_x = lax.axis_index("x")
        my_y = lax.axis_index("y")
        peer_y = 1 - my_y

        barrier = pltpu.get_barrier_semaphore()
        pl.semaphore_signal(
            barrier, inc=1,
            device_id=(my_x, peer_y), device_id_type=pl.DeviceIdType.MESH,
        )
        pl.semaphore_wait(barrier, 1)

        wo = wo_ref[:, :]

        for b in range(B):
            o_peer = o_ref[b, pl.ds(peer_y * S_half, S_half), :]
            sbuf[b] = jnp.dot(o_peer, wo, preferred_element_type=jnp.float32)

        rdma = pltpu.make_async_remote_copy(
            src_ref=sbuf,
            dst_ref=rbuf,
            send_sem=send_sem,
            recv_sem=recv_sem,
            device_id=(my_x, peer_y),
            device_id_type=pl.DeviceIdType.MESH,
        )
        rdma.start()

        for b in range(B):
            o_own = o_ref[b, pl.ds(my_y * S_half, S_half), :]
            out_ref[b] = jnp.dot(o_own, wo, preferred_element_type=jnp.float32)

        rdma.wait()
        out_ref[:, :, :] += rbuf[:, :, :]

    return pl.pallas_call(
        body,
        out_shape=jax.ShapeDtypeStruct((B, S_half, N), jnp.float32),
        in_specs=[
            pl.BlockSpec(memory_space=pltpu.VMEM),
            pl.BlockSpec(memory_space=pltpu.VMEM),
        ],
        out_specs=pl.BlockSpec(memory_space=pltpu.VMEM),
        scratch_shapes=[
            pltpu.VMEM((B, S_half, N), jnp.float32),
            pltpu.VMEM((B, S_half, N), jnp.float32),
            pltpu.SemaphoreType.DMA,
            pltpu.SemaphoreType.DMA,
        ],
        compiler_params=pltpu.CompilerParams(collective_id=0),
    )(O2, Wo)


# device time: 108038 ns/iter; 1.0437x vs baseline; 1.0437x over previous
import jax
import jax.numpy as jnp
from jax import lax
from jax.experimental import pallas as pl
from jax.experimental.pallas import tpu as pltpu


def kernel(O, Wo):
    B, S, H, D = O.shape
    HD = H * D
    N = Wo.shape[1]
    S_half = S // 2

    O2 = O.reshape(B, S, HD)

    CHUNK = 128
    n_chunks_per_b = S_half // CHUNK
    n_chunks = B * n_chunks_per_b

    def body(o_ref, wo_ref, out_ref, sbuf, rbuf, send_sems, recv_sems):
        my_x = lax.axis_index("x")
        my_y = lax.axis_index("y")
        peer_y = 1 - my_y

        barrier = pltpu.get_barrier_semaphore()
        pl.semaphore_signal(
            barrier, inc=1,
            device_id=(my_x, peer_y), device_id_type=pl.DeviceIdType.MESH,
        )
        pl.semaphore_wait(barrier, 1)

        wo = wo_ref[:, :]

        def chunk_rdma(c):
            b, j = divmod(c, n_chunks_per_b)
            return pltpu.make_async_remote_copy(
                src_ref=sbuf.at[b, pl.ds(j * CHUNK, CHUNK), :],
                dst_ref=rbuf.at[b, pl.ds(j * CHUNK, CHUNK), :],
                send_sem=send_sems.at[c],
                recv_sem=recv_sems.at[c],
                device_id=(my_x, peer_y),
                device_id_type=pl.DeviceIdType.MESH,
            )

        rdmas = []
        for c in range(n_chunks):
            b, j = divmod(c, n_chunks_per_b)
            o_c = o_ref[b, pl.ds(peer_y * S_half + j * CHUNK, CHUNK), :]
            sbuf[b, pl.ds(j * CHUNK, CHUNK), :] = jnp.dot(
                o_c, wo, preferred_element_type=jnp.float32
            )
            r = chunk_rdma(c)
            r.start()
            rdmas.append(r)

        for b in range(B):
            o_own = o_ref[b, pl.ds(my_y * S_half, S_half), :]
            out_ref[b] = jnp.dot(o_own, wo, preferred_element_type=jnp.float32)

        for c in range(n_chunks):
            b, j = divmod(c, n_chunks_per_b)
            rdmas[c].wait_recv()
            out_ref[b, pl.ds(j * CHUNK, CHUNK), :] += rbuf[
                b, pl.ds(j * CHUNK, CHUNK), :
            ]
        for c in range(n_chunks):
            rdmas[c].wait_send()

    return pl.pallas_call(
        body,
        out_shape=jax.ShapeDtypeStruct((B, S_half, N), jnp.float32),
        in_specs=[
            pl.BlockSpec(memory_space=pltpu.VMEM),
            pl.BlockSpec(memory_space=pltpu.VMEM),
        ],
        out_specs=pl.BlockSpec(memory_space=pltpu.VMEM),
        scratch_shapes=[
            pltpu.VMEM((B, S_half, N), jnp.float32),
            pltpu.VMEM((B, S_half, N), jnp.float32),
            pltpu.SemaphoreType.DMA((n_chunks,)),
            pltpu.SemaphoreType.DMA((n_chunks,)),
        ],
        compiler_params=pltpu.CompilerParams(collective_id=0),
    )(O2, Wo)


# device time: 76685 ns/iter; 1.4704x vs baseline; 1.4089x over previous
import jax
import jax.numpy as jnp
from jax import lax
from jax.experimental import pallas as pl
from jax.experimental.pallas import tpu as pltpu


def kernel(O, Wo):
    B, S, H, D = O.shape
    HD = H * D
    N = Wo.shape[1]
    S_half = S // 2
    Q = S_half // 2

    O2 = O.reshape(B, S, HD)

    CHUNK = 64
    n_j = Q // CHUNK
    n_chunks = B * n_j

    def body(o_ref, wo_ref, out_ref, y_sbuf, y_rbuf, x_sbuf, x_rbuf,
             y_send_sems, y_recv_sems, x_send_sems, x_recv_sems):
        my_x = lax.axis_index("x")
        my_y = lax.axis_index("y")
        peer_y = 1 - my_y
        peer_x = 1 - my_x

        barrier = pltpu.get_barrier_semaphore()
        pl.semaphore_signal(
            barrier, inc=1,
            device_id=(my_x, peer_y), device_id_type=pl.DeviceIdType.MESH,
        )
        pl.semaphore_signal(
            barrier, inc=1,
            device_id=(peer_x, my_y), device_id_type=pl.DeviceIdType.MESH,
        )
        pl.semaphore_wait(barrier, 2)

        wo = wo_ref[:, :]

        y_rdmas = []
        for c in range(n_chunks):
            b, j = divmod(c, n_j)
            row0 = peer_y * S_half + my_x * Q + j * CHUNK
            y_sbuf[b, pl.ds(j * CHUNK, CHUNK), :] = jnp.dot(
                o_ref[b, pl.ds(row0, CHUNK), :], wo,
                preferred_element_type=jnp.float32,
            )
            r = pltpu.make_async_remote_copy(
                src_ref=y_sbuf.at[b, pl.ds(j * CHUNK, CHUNK), :],
                dst_ref=y_rbuf.at[b, pl.ds(j * CHUNK, CHUNK), :],
                send_sem=y_send_sems.at[c],
                recv_sem=y_recv_sems.at[c],
                device_id=(my_x, peer_y),
                device_id_type=pl.DeviceIdType.MESH,
            )
            r.start()
            y_rdmas.append(r)

        for c in range(n_chunks):
            b, j = divmod(c, n_j)
            row0 = my_y * S_half + my_x * Q + j * CHUNK
            out_ref[b, pl.ds(my_x * Q + j * CHUNK, CHUNK), :] = jnp.dot(
                o_ref[b, pl.ds(row0, CHUNK), :], wo,
                preferred_element_type=jnp.float32,
            )

        x_rdmas = []
        for c in range(n_chunks):
            b, j = divmod(c, n_j)
            out_row0 = my_x * Q + j * CHUNK
            y_rdmas[c].wait_recv()
            red = (
                out_ref[b, pl.ds(out_row0, CHUNK), :]
                + y_rbuf[b, pl.ds(j * CHUNK, CHUNK), :]
            )
            out_ref[b, pl.ds(out_row0, CHUNK), :] = red
            x_sbuf[b, pl.ds(j * CHUNK, CHUNK), :] = red
            r = pltpu.make_async_remote_copy(
                src_ref=x_sbuf.at[b, pl.ds(j * CHUNK, CHUNK), :],
                dst_ref=x_rbuf.at[b, pl.ds(j * CHUNK, CHUNK), :],
                send_sem=x_send_sems.at[c],
                recv_sem=x_recv_sems.at[c],
                device_id=(peer_x, my_y),
                device_id_type=pl.DeviceIdType.MESH,
            )
            r.start()
            x_rdmas.append(r)

        for c in range(n_chunks):
            b, j = divmod(c, n_j)
            x_rdmas[c].wait_recv()
            out_ref[b, pl.ds(peer_x * Q + j * CHUNK, CHUNK), :] = x_rbuf[
                b, pl.ds(j * CHUNK, CHUNK), :
            ]
        for c in range(n_chunks):
            y_rdmas[c].wait_send()
            x_rdmas[c].wait_send()

    return pl.pallas_call(
        body,
        out_shape=jax.ShapeDtypeStruct((B, S_half, N), jnp.float32),
        in_specs=[
            pl.BlockSpec(memory_space=pltpu.VMEM),
            pl.BlockSpec(memory_space=pltpu.VMEM),
        ],
        out_specs=pl.BlockSpec(memory_space=pltpu.VMEM),
        scratch_shapes=[
            pltpu.VMEM((B, Q, N), jnp.float32),
            pltpu.VMEM((B, Q, N), jnp.float32),
            pltpu.VMEM((B, Q, N), jnp.float32),
            pltpu.VMEM((B, Q, N), jnp.float32),
            pltpu.SemaphoreType.DMA((n_chunks,)),
            pltpu.SemaphoreType.DMA((n_chunks,)),
            pltpu.SemaphoreType.DMA((n_chunks,)),
            pltpu.SemaphoreType.DMA((n_chunks,)),
        ],
        compiler_params=pltpu.CompilerParams(collective_id=0),
    )(O2, Wo)
